# baseline (speedup 1.0000x reference)
import jax
import jax.numpy as jnp
from jax.experimental import pallas as pl
from jax.experimental.pallas import tpu as pltpu

_B, _N = 32, 1024
_F, _E, _U, _H = 8, 8, 32, 6
_TPAD = 128


def _gcn_body(pdg_ref, feat_ref, adj_ref, embt_ref, wf_ref, we_ref, bin_ref,
              wh_ref, bh_ref, wout_ref, bout_ref, out_ref):
    adjm = adj_ref[0]
    featm = feat_ref[0]
    ids = pdg_ref[0]

    cols = jax.lax.broadcasted_iota(jnp.int32, (_N, _TPAD), 1)
    onehot = (cols == ids).astype(jnp.float32)
    emb = jnp.dot(onehot, embt_ref[...],
                  preferred_element_type=jnp.float32)

    h = jnp.dot(featm, wf_ref[...], preferred_element_type=jnp.float32)
    h = h + jnp.dot(emb, we_ref[...], preferred_element_type=jnp.float32)
    h = jnp.maximum(h + bin_ref[...], 0.0)

    deg = jnp.sum(adjm, axis=1, keepdims=True)
    inv = 1.0 / (deg + 1e-8)

    adjb = adjm.astype(jnp.bfloat16)

    for i in range(_H):
        m = jnp.dot(adjb, h.astype(jnp.bfloat16),
                    preferred_element_type=jnp.float32) * inv
        h = jnp.dot(m, wh_ref[i], preferred_element_type=jnp.float32)
        h = jnp.maximum(h + bh_ref[i], 0.0)

    pooled = jnp.sum(h, axis=0, keepdims=True)
    res = jnp.dot(pooled, wout_ref[...],
                  preferred_element_type=jnp.float32) + bout_ref[...]
    out_ref[0] = jnp.broadcast_to(res, (1, 128))


def kernel(pdg, feat, adj, emb_table, W_in, b_in, W_h, b_h, W_out, b_out):
    pdg3 = pdg.astype(jnp.int32).reshape(_B, _N, 1)
    embp = jnp.zeros((_TPAD, _E), jnp.float32).at[:emb_table.shape[0]].set(
        emb_table.astype(jnp.float32))
    wf = W_in[:_F]
    we = W_in[_F:]
    bin2 = b_in.reshape(1, _U)
    bh3 = b_h.reshape(_H, 1, _U)
    bout2 = b_out.reshape(1, 1)

    out = pl.pallas_call(
        _gcn_body,
        grid=(_B,),
        in_specs=[
            pl.BlockSpec((1, _N, 1), lambda b: (b, 0, 0)),
            pl.BlockSpec((1, _N, _F), lambda b: (b, 0, 0)),
            pl.BlockSpec((1, _N, _N), lambda b: (b, 0, 0)),
            pl.BlockSpec((_TPAD, _E), lambda b: (0, 0)),
            pl.BlockSpec((_F, _U), lambda b: (0, 0)),
            pl.BlockSpec((_E, _U), lambda b: (0, 0)),
            pl.BlockSpec((1, _U), lambda b: (0, 0)),
            pl.BlockSpec((_H, _U, _U), lambda b: (0, 0, 0)),
            pl.BlockSpec((_H, 1, _U), lambda b: (0, 0, 0)),
            pl.BlockSpec((_U, 1), lambda b: (0, 0)),
            pl.BlockSpec((1, 1), lambda b: (0, 0)),
        ],
        out_specs=pl.BlockSpec((1, 1, 128), lambda b: (b, 0, 0)),
        out_shape=jax.ShapeDtypeStruct((_B, 1, 128), jnp.float32),
        compiler_params=pltpu.CompilerParams(
            dimension_semantics=("parallel",)),
    )(pdg3, feat, adj, embp, wf, we, bin2, W_h, bh3, W_out, bout2)
    return out[:, 0, :1]


# speedup vs baseline: 1.1650x; 1.1650x over previous
import jax
import jax.numpy as jnp
from jax.experimental import pallas as pl
from jax.experimental.pallas import tpu as pltpu

_B, _N = 32, 1024
_F, _E, _U, _H = 8, 8, 32, 6
_G = 4
_CHUNK = 256
_TPAD = 128


def _gcn_body(pdg_ref, feat_ref, adj_ref, embt_ref, wf_ref, we_ref, bin_ref,
              wh_ref, bh_ref, wout_ref, bout_ref, out_ref):
    adjb, inv, h = [], [], []
    for g in range(_G):
        adjm = adj_ref[g]
        featm = feat_ref[g]
        ids = pdg_ref[g]

        cols = jax.lax.broadcasted_iota(jnp.int32, (_N, _TPAD), 1)
        onehot = (cols == ids).astype(jnp.float32)
        emb = jnp.dot(onehot, embt_ref[...],
                      preferred_element_type=jnp.float32)

        hg = jnp.dot(featm, wf_ref[...], preferred_element_type=jnp.float32)
        hg = hg + jnp.dot(emb, we_ref[...], preferred_element_type=jnp.float32)
        h.append(jnp.maximum(hg + bin_ref[...], 0.0))

        deg = jnp.sum(adjm, axis=1, keepdims=True)
        inv.append(1.0 / (deg + 1e-8))

        adjb.append(adjm.astype(jnp.bfloat16))

    nc = _N // _CHUNK
    for i in range(_H):
        hb = [h[g].astype(jnp.bfloat16) for g in range(_G)]
        pieces = [[] for _ in range(_G)]
        for c in range(nc):
            lo = c * _CHUNK
            for g in range(_G):
                m = jnp.dot(adjb[g][lo:lo + _CHUNK, :], hb[g],
                            preferred_element_type=jnp.float32)
                m = m * inv[g][lo:lo + _CHUNK, :]
                hg = jnp.dot(m, wh_ref[i], preferred_element_type=jnp.float32)
                pieces[g].append(jnp.maximum(hg + bh_ref[i], 0.0))
        h = [jnp.concatenate(pieces[g], axis=0) for g in range(_G)]

    for g in range(_G):
        pooled = jnp.sum(h[g], axis=0, keepdims=True)
        res = jnp.dot(pooled, wout_ref[...],
                      preferred_element_type=jnp.float32) + bout_ref[...]
        out_ref[g] = jnp.broadcast_to(res, (1, 128))


def kernel(pdg, feat, adj, emb_table, W_in, b_in, W_h, b_h, W_out, b_out):
    pdg3 = pdg.astype(jnp.int32).reshape(_B, _N, 1)
    embp = jnp.zeros((_TPAD, _E), jnp.float32).at[:emb_table.shape[0]].set(
        emb_table.astype(jnp.float32))
    wf = W_in[:_F]
    we = W_in[_F:]
    bin2 = b_in.reshape(1, _U)
    bh3 = b_h.reshape(_H, 1, _U)
    bout2 = b_out.reshape(1, 1)

    out = pl.pallas_call(
        _gcn_body,
        grid=(_B // _G,),
        in_specs=[
            pl.BlockSpec((_G, _N, 1), lambda b: (b, 0, 0)),
            pl.BlockSpec((_G, _N, _F), lambda b: (b, 0, 0)),
            pl.BlockSpec((_G, _N, _N), lambda b: (b, 0, 0)),
            pl.BlockSpec((_TPAD, _E), lambda b: (0, 0)),
            pl.BlockSpec((_F, _U), lambda b: (0, 0)),
            pl.BlockSpec((_E, _U), lambda b: (0, 0)),
            pl.BlockSpec((1, _U), lambda b: (0, 0)),
            pl.BlockSpec((_H, _U, _U), lambda b: (0, 0, 0)),
            pl.BlockSpec((_H, 1, _U), lambda b: (0, 0, 0)),
            pl.BlockSpec((_U, 1), lambda b: (0, 0)),
            pl.BlockSpec((1, 1), lambda b: (0, 0)),
        ],
        out_specs=pl.BlockSpec((_G, 1, 128), lambda b: (b, 0, 0)),
        out_shape=jax.ShapeDtypeStruct((_B, 1, 128), jnp.float32),
        compiler_params=pltpu.CompilerParams(
            dimension_semantics=("arbitrary",)),
    )(pdg3, feat, adj, embp, wf, we, bin2, W_h, bh3, W_out, bout2)
    return out[:, 0, :1]


# speedup vs baseline: 2.7992x; 2.4028x over previous
import jax
import jax.numpy as jnp
from jax.experimental import pallas as pl
from jax.experimental.pallas import tpu as pltpu

_B, _N = 32, 1024
_F, _E, _U, _H = 8, 8, 32, 6
_G = 4
_CHUNK = 256
_TPAD = 128


def _gcn_body(pdg_ref, featT_ref, adj_ref, embtT_ref, wfT_ref, weT_ref,
              binT_ref, whT_ref, bhT_ref, woutT_ref, bout_ref, out_ref,
              adjT_ref):
    adjT, invT, h = [], [], []
    ones_row = jnp.ones((1, _N), jnp.bfloat16)
    for g in range(_G):
        featT = featT_ref[g]
        ids = pdg_ref[g]

        rows = jax.lax.broadcasted_iota(jnp.int32, (_TPAD, _N), 0)
        onehotT = (rows == ids).astype(jnp.float32)
        embT = jnp.dot(embtT_ref[...], onehotT,
                       preferred_element_type=jnp.float32)

        hg = jnp.dot(wfT_ref[...], featT, preferred_element_type=jnp.float32)
        hg = hg + jnp.dot(weT_ref[...], embT,
                          preferred_element_type=jnp.float32)
        h.append(jnp.maximum(hg + binT_ref[...], 0.0))

        adjT_ref[g] = jnp.transpose(adj_ref[g].astype(jnp.bfloat16))
        aT = adjT_ref[g]
        adjT.append(aT)

        degT = jnp.dot(ones_row, aT, preferred_element_type=jnp.float32)
        invT.append(1.0 / (degT + 1e-8))

    for i in range(_H):
        mTs = [jnp.dot(h[g].astype(jnp.bfloat16), adjT[g],
                       preferred_element_type=jnp.float32)
               for g in range(_G)]
        for g in range(_G):
            mT = mTs[g] * invT[g]
            hg = jnp.dot(whT_ref[i], mT, preferred_element_type=jnp.float32)
            h[g] = jnp.maximum(hg + bhT_ref[i], 0.0)

    for g in range(_G):
        pooledT = jnp.sum(h[g], axis=1, keepdims=True)
        res = jnp.dot(woutT_ref[...], pooledT,
                      preferred_element_type=jnp.float32) + bout_ref[...]
        out_ref[g] = jnp.broadcast_to(res, (1, 128))


def kernel(pdg, feat, adj, emb_table, W_in, b_in, W_h, b_h, W_out, b_out):
    pdg3 = pdg.astype(jnp.int32).reshape(_B, 1, _N)
    featT = jnp.swapaxes(feat, 1, 2)
    embtT = jnp.zeros((_E, _TPAD), jnp.float32).at[:, :emb_table.shape[0]].set(
        emb_table.astype(jnp.float32).T)
    wfT = W_in[:_F].T
    weT = W_in[_F:].T
    binT = b_in.reshape(_U, 1)
    whT = jnp.swapaxes(W_h, 1, 2)
    bhT = b_h.reshape(_H, _U, 1)
    woutT = W_out.T
    bout2 = b_out.reshape(1, 1)

    out = pl.pallas_call(
        _gcn_body,
        grid=(_B // _G,),
        in_specs=[
            pl.BlockSpec((_G, 1, _N), lambda b: (b, 0, 0)),
            pl.BlockSpec((_G, _F, _N), lambda b: (b, 0, 0)),
            pl.BlockSpec((_G, _N, _N), lambda b: (b, 0, 0)),
            pl.BlockSpec((_E, _TPAD), lambda b: (0, 0)),
            pl.BlockSpec((_U, _F), lambda b: (0, 0)),
            pl.BlockSpec((_U, _E), lambda b: (0, 0)),
            pl.BlockSpec((_U, 1), lambda b: (0, 0)),
            pl.BlockSpec((_H, _U, _U), lambda b: (0, 0, 0)),
            pl.BlockSpec((_H, _U, 1), lambda b: (0, 0, 0)),
            pl.BlockSpec((1, _U), lambda b: (0, 0)),
            pl.BlockSpec((1, 1), lambda b: (0, 0)),
        ],
        out_specs=pl.BlockSpec((_G, 1, 128), lambda b: (b, 0, 0)),
        out_shape=jax.ShapeDtypeStruct((_B, 1, 128), jnp.float32),
        scratch_shapes=[pltpu.VMEM((_G, _N, _N), jnp.bfloat16)],
        compiler_params=pltpu.CompilerParams(
            dimension_semantics=("arbitrary",)),
    )(pdg3, featT, adj, embtT, wfT, weT, binT, whT, bhT, woutT, bout2)
    return out[:, 0, :1]


# speedup vs baseline: 2.8243x; 1.0090x over previous
import jax
import jax.numpy as jnp
from jax.experimental import pallas as pl
from jax.experimental.pallas import tpu as pltpu

_B, _N = 32, 1024
_F, _E, _U, _H = 8, 8, 32, 6
_G = 4
_TPAD = 128


def _gcn_body(pdg_ref, featT_ref, adj_ref, embtT_ref, wfT_ref, weT_ref,
              binT_ref, whT_ref, bhT_ref, woutT_ref, bout_ref, out_ref,
              adjT_ref):
    adjT, h = [], []
    for g in range(_G):
        featT = featT_ref[g]
        ids = pdg_ref[g]

        rows = jax.lax.broadcasted_iota(jnp.int32, (_TPAD, _N), 0)
        onehotT = (rows == ids).astype(jnp.float32)
        embT = jnp.dot(embtT_ref[...], onehotT,
                       preferred_element_type=jnp.float32)

        hg = jnp.dot(wfT_ref[...], featT, preferred_element_type=jnp.float32)
        hg = hg + jnp.dot(weT_ref[...], embT,
                          preferred_element_type=jnp.float32)
        h.append(jnp.maximum(hg + binT_ref[...], 0.0))

        adjm = adj_ref[g]
        deg = jnp.sum(adjm, axis=1, keepdims=True)
        anorm = adjm * (1.0 / (deg + 1e-8))
        adjT_ref[g] = jnp.transpose(anorm.astype(jnp.bfloat16))
        adjT.append(adjT_ref[g])

    for i in range(_H):
        mTs = [jnp.dot(h[g].astype(jnp.bfloat16), adjT[g],
                       preferred_element_type=jnp.float32)
               for g in range(_G)]
        for g in range(_G):
            hg = jnp.dot(whT_ref[i], mTs[g],
                         preferred_element_type=jnp.float32)
            h[g] = jnp.maximum(hg + bhT_ref[i], 0.0)

    for g in range(_G):
        pooledT = jnp.sum(h[g], axis=1, keepdims=True)
        res = jnp.dot(woutT_ref[...], pooledT,
                      preferred_element_type=jnp.float32) + bout_ref[...]
        out_ref[g] = jnp.broadcast_to(res, (1, 128))


def kernel(pdg, feat, adj, emb_table, W_in, b_in, W_h, b_h, W_out, b_out):
    pdg3 = pdg.astype(jnp.int32).reshape(_B, 1, _N)
    featT = jnp.swapaxes(feat, 1, 2)
    embtT = jnp.zeros((_E, _TPAD), jnp.float32).at[:, :emb_table.shape[0]].set(
        emb_table.astype(jnp.float32).T)
    wfT = W_in[:_F].T
    weT = W_in[_F:].T
    binT = b_in.reshape(_U, 1)
    whT = jnp.swapaxes(W_h, 1, 2)
    bhT = b_h.reshape(_H, _U, 1)
    woutT = W_out.T
    bout2 = b_out.reshape(1, 1)

    out = pl.pallas_call(
        _gcn_body,
        grid=(_B // _G,),
        in_specs=[
            pl.BlockSpec((_G, 1, _N), lambda b: (b, 0, 0)),
            pl.BlockSpec((_G, _F, _N), lambda b: (b, 0, 0)),
            pl.BlockSpec((_G, _N, _N), lambda b: (b, 0, 0)),
            pl.BlockSpec((_E, _TPAD), lambda b: (0, 0)),
            pl.BlockSpec((_U, _F), lambda b: (0, 0)),
            pl.BlockSpec((_U, _E), lambda b: (0, 0)),
            pl.BlockSpec((_U, 1), lambda b: (0, 0)),
            pl.BlockSpec((_H, _U, _U), lambda b: (0, 0, 0)),
            pl.BlockSpec((_H, _U, 1), lambda b: (0, 0, 0)),
            pl.BlockSpec((1, _U), lambda b: (0, 0)),
            pl.BlockSpec((1, 1), lambda b: (0, 0)),
        ],
        out_specs=pl.BlockSpec((_G, 1, 128), lambda b: (b, 0, 0)),
        out_shape=jax.ShapeDtypeStruct((_B, 1, 128), jnp.float32),
        scratch_shapes=[pltpu.VMEM((_G, _N, _N), jnp.bfloat16)],
        compiler_params=pltpu.CompilerParams(
            dimension_semantics=("arbitrary",)),
    )(pdg3, featT, adj, embtT, wfT, weT, binT, whT, bhT, woutT, bout2)
    return out[:, 0, :1]
